# baseline (speedup 1.0000x reference)
import functools

import jax
import jax.numpy as jnp
from jax import lax
from jax.experimental import pallas as pl
from jax.experimental.pallas import tpu as pltpu
from jax.experimental.pallas import tpu_sc as plsc

_VOCAB = 8192
_BATCH = 4096
_D = 8192

_info = plsc.get_sparse_core_info()
_NC = _info.num_cores
_NS = _info.num_subcores
_NW = _NC * _NS
_BPW = _BATCH // _NW
_R = 2
_CH = _BPW // _R
_NB = 4

_mesh = plsc.VectorSubcoreMesh(core_axis_name="c", subcore_axis_name="s")


@functools.partial(
    pl.kernel,
    mesh=_mesh,
    out_type=jax.ShapeDtypeStruct((_BATCH, _D), jnp.float32),
    scratch_types=[
        pltpu.VMEM((_CH, _R), jnp.int32),
        pltpu.VMEM((_NB, _R, _D), jnp.float32),
    ] + [pltpu.SemaphoreType.DMA] * (2 * _NB),
)
def _sc_gather(idx_hbm, table_hbm, out_hbm, idx_v, bufs, *sems):
    gs = sems[:_NB]
    ws = sems[_NB:]
    wid = lax.axis_index("s") * _NC + lax.axis_index("c")
    base = wid * _BPW
    pltpu.sync_copy(idx_hbm.at[wid], idx_v)

    for b in range(2):
        pltpu.async_copy(table_hbm.at[idx_v.at[b]], bufs.at[b], gs[b])

    def step(g, carry):
        for b in range(_NB):
            c = g * _NB + b
            pltpu.make_async_copy(table_hbm.at[idx_v.at[c]], bufs.at[b], gs[b]).wait()
            pltpu.async_copy(bufs.at[b], out_hbm.at[pl.ds(base + c * _R, _R)], ws[b])
            nb = (b + 2) % _NB

            @pl.when(c >= 2)
            def _reclaim():
                pltpu.make_async_copy(
                    bufs.at[nb], out_hbm.at[pl.ds(base + (c - 2) * _R, _R)], ws[nb]
                ).wait()

            @pl.when(c + 2 < _CH)
            def _launch_next():
                pltpu.async_copy(table_hbm.at[idx_v.at[c + 2]], bufs.at[nb], gs[nb])

        return carry

    lax.fori_loop(0, _CH // _NB, step, 0)

    for c in (_CH - 2, _CH - 1):
        b = c % _NB
        pltpu.make_async_copy(
            bufs.at[b], out_hbm.at[pl.ds(base + c * _R, _R)], ws[b]
        ).wait()


def kernel(x, table):
    idx = x.astype(jnp.int32).reshape(_NW, _CH, _R)
    return _sc_gather(idx, table)
